# baseline (speedup 1.0000x reference)
import functools
import math

import jax
import jax.numpy as jnp
from jax import lax
from jax.experimental import pallas as pl
from jax.experimental.pallas import tpu as pltpu
from jax.experimental.pallas import tpu_sc as plsc

DM = 64
PAD = 128
SCALE = math.sqrt(DM)

NC = 2
NS = 16
NW = NC * NS
L = 16


def _emb_lookup(tp, x):
    nb, sl = x.shape
    rows_per_w = nb // NW
    assert rows_per_w % 2 == 0
    g0 = 128
    g1 = sl - g0

    mesh = plsc.VectorSubcoreMesh(core_axis_name="c", subcore_axis_name="s")

    @functools.partial(
        pl.kernel,
        mesh=mesh,
        out_type=jax.ShapeDtypeStruct((nb, sl, DM), jnp.float32),
        scratch_types=[
            pltpu.VMEM((2, sl), jnp.int32),
            pltpu.VMEM((2, sl, PAD), jnp.float32),
            pltpu.VMEM((2, 1, sl, DM), jnp.float32),
            pltpu.SemaphoreType.DMA,
            pltpu.SemaphoreType.DMA,
            pltpu.SemaphoreType.DMA,
            pltpu.SemaphoreType.DMA,
        ],
        compiler_params=pltpu.CompilerParams(use_tc_tiling_on_sc=True),
    )
    def k(tp_hbm, x_hbm, out_hbm, idx_v, g_v, o_v, gsem0, gsem1, wsem0,
          wsem1):
        gsems = (gsem0, gsem1)
        wsems = (wsem0, wsem1)
        wid = lax.axis_index("s") * NC + lax.axis_index("c")
        base = wid * rows_per_w

        def fire(c, bb):
            pltpu.sync_copy(x_hbm.at[base + c], idx_v.at[bb])
            pltpu.async_copy(
                tp_hbm.at[idx_v.at[bb, pl.ds(0, g0)]],
                g_v.at[bb, pl.ds(0, g0)],
                gsems[bb],
            )
            pltpu.async_copy(
                tp_hbm.at[idx_v.at[bb, pl.ds(g0, g1)]],
                g_v.at[bb, pl.ds(g0, g1)],
                gsems[bb],
            )

        def drain_g(bb):
            pltpu.make_async_copy(tp_hbm.at[pl.ds(0, sl)],
                                  g_v.at[bb], gsems[bb]).wait()

        def drain_w(bb):
            pltpu.make_async_copy(out_hbm.at[pl.ds(0, 1)],
                                  o_v.at[bb], wsems[bb]).wait()

        fire(0, 0)

        def pair(t, carry):
            go = t * 2
            for b in (0, 1):
                c = go + b
                nb_ = 1 - b

                @pl.when(c + 1 < rows_per_w)
                def _():
                    @pl.when(c >= 1)
                    def _():
                        drain_w(nb_)
                    fire(c + 1, nb_)

                drain_g(b)

                def scale_row(i, cr, _b=b):
                    for j in range(DM // L):
                        s = pl.ds(j * L, L)
                        o_v[_b, 0, i, s] = g_v[_b, i, s] * SCALE
                    return cr

                lax.fori_loop(0, sl, scale_row, 0, unroll=4)
                pltpu.async_copy(
                    o_v.at[b],
                    out_hbm.at[pl.ds(base + c, 1)],
                    wsems[b],
                )
            return carry

        lax.fori_loop(0, rows_per_w // 2, pair, 0)
        drain_w(0)
        drain_w(1)

    return k(tp, x)


def kernel(x, table):
    tp = jnp.pad(table, ((0, 0), (0, PAD - DM)))
    return _emb_lookup(tp, x)
